# baseline (speedup 1.0000x reference)
import functools

import jax
import jax.numpy as jnp
from jax import lax
from jax.experimental import pallas as pl
from jax.experimental.pallas import tpu as pltpu
from jax.experimental.pallas import tpu_sc as plsc

NUM_NODES = 100000
MEM_DIM = 128
B = 1024
U = 2 * B

_HIGH = jax.lax.Precision.HIGHEST


def _dot(a, b):
    return jax.lax.dot_general(a, b, (((1,), (0,)), ((), ())),
                               precision=_HIGH, preferred_element_type=jnp.float32)


LU_ROWS = 784


def _tc_chain_body(rows_ref, edge_ref, ts_ref, lu_ref, nodes_c_ref, nodes_r_ref,
                   pos_c_ref, pos_r_ref, w1s_ref, w1d_ref, w1e_ref, b1_ref,
                   w2_ref, b2_ref, wih_ref, bih_ref, whh_ref, bhh_ref,
                   out_rows_ref, out_idx_ref, out_lu_ref):
    rows = rows_ref[...]
    src_mem = rows[:B]
    dst_mem = rows[B:]

    pre = (_dot(src_mem, w1s_ref[...]) + _dot(dst_mem, w1d_ref[...])
           + _dot(edge_ref[...], w1e_ref[...]) + b1_ref[...])
    msg = _dot(jnp.maximum(pre, 0.0), w2_ref[...]) + b2_ref[...]

    gi = _dot(msg, wih_ref[...]) + bih_ref[...]
    gi2 = jnp.concatenate([gi, gi], axis=0)
    i_r = gi2[:, :MEM_DIM]
    i_z = gi2[:, MEM_DIM:2 * MEM_DIM]
    i_n = gi2[:, 2 * MEM_DIM:]

    nodes_c = nodes_c_ref[...]
    nodes_r = nodes_r_ref[...]
    pos_c = pos_c_ref[...]
    pos_r = pos_r_ref[...]
    eq = nodes_c == nodes_r

    big = jnp.int32(1 << 30)
    succ = jnp.where(eq & (pos_c > pos_r), pos_c, big)
    next_pos = jnp.min(succ, axis=0, keepdims=True)
    route = (pos_c == next_pos).astype(jnp.float32)
    incoming = jnp.max(route, axis=1, keepdims=True)

    cnt = jnp.sum(eq.astype(jnp.int32), axis=0)
    rounds = jnp.max(cnt)

    last_pos = jnp.max(jnp.where(eq, pos_r, -1), axis=1, keepdims=True)
    is_last = (last_pos == pos_c).astype(jnp.float32)

    whh = whh_ref[...]
    bhh = bhh_ref[...]

    def gru(h):
        gh = _dot(h, whh) + bhh
        h_r = gh[:, :MEM_DIM]
        h_z = gh[:, MEM_DIM:2 * MEM_DIM]
        h_n = gh[:, 2 * MEM_DIM:]
        r = jax.nn.sigmoid(i_r + h_r)
        z = jax.nn.sigmoid(i_z + h_z)
        n = jnp.tanh(i_n + r * h_n)
        return (1.0 - z) * n + z * h

    def body(_, h):
        g = gru(h)
        return _dot(route, g) + (1.0 - incoming) * h

    h = lax.fori_loop(0, rounds - 1, body, rows)
    g = gru(h)
    out_rows_ref[...] = g
    out_idx_ref[...] = jnp.where(is_last > 0.5, nodes_c, -1)

    lane = lax.broadcasted_iota(jnp.int32, (1, MEM_DIM), 1)
    lane_oh = (lane == nodes_c % MEM_DIM).astype(jnp.float32) * is_last
    rowid = lax.broadcasted_iota(jnp.int32, (LU_ROWS, 1), 0)
    q_row = nodes_r // MEM_DIM
    rowpick = (rowid == q_row).astype(jnp.float32)
    lu_new = _dot(rowpick, lane_oh * ts_ref[...])
    touched = _dot(rowpick, lane_oh)
    out_lu_ref[...] = jnp.where(touched > 0.5, lu_new, lu_ref[...])


def _tc_chain(rows, edge_p, ts_col, lu2d, nodes_c, nodes_r, pos_c, pos_r,
              w1s, w1d, w1e, b1, w2, b2, wih, bih, whh, bhh):
    return pl.pallas_call(
        _tc_chain_body,
        out_shape=(jax.ShapeDtypeStruct((U, MEM_DIM), jnp.float32),
                   jax.ShapeDtypeStruct((U, 1), jnp.int32),
                   jax.ShapeDtypeStruct((LU_ROWS, MEM_DIM), jnp.float32)),
    )(rows, edge_p, ts_col, lu2d, nodes_c, nodes_r, pos_c, pos_r,
      w1s, w1d, w1e, b1, w2, b2, wih, bih, whh, bhh)


@functools.lru_cache(maxsize=None)
def _sc_kernels():
    mesh = plsc.VectorSubcoreMesh(core_axis_name="c", subcore_axis_name="s")
    nc = mesh.num_cores
    nw = nc * mesh.num_subcores
    bpw = U // nw

    def _wid():
        return lax.axis_index("s") * nc + lax.axis_index("c")

    @functools.partial(
        pl.kernel, mesh=mesh,
        out_type=jax.ShapeDtypeStruct((U, MEM_DIM), jnp.float32),
        scratch_types=[pltpu.VMEM((bpw,), jnp.int32),
                       pltpu.VMEM((bpw, MEM_DIM), jnp.float32),
                       pltpu.SemaphoreType.DMA],
    )
    def gather_k(table_hbm, idx_hbm, out_hbm, idx_v, rows_v, sem):
        base = _wid() * bpw
        pltpu.sync_copy(idx_hbm.at[pl.ds(base, bpw)], idx_v)
        pltpu.async_copy(table_hbm.at[idx_v], rows_v, sem).wait()
        pltpu.sync_copy(rows_v, out_hbm.at[pl.ds(base, bpw)])

    rng = 3128
    last_rng = NUM_NODES - 31 * rng

    @functools.partial(
        pl.kernel, mesh=mesh,
        out_type=jax.ShapeDtypeStruct((NUM_NODES, MEM_DIM), jnp.float32),
    )
    def copy_k(src_hbm, out_hbm):
        wid = _wid()
        base = pl.multiple_of(wid * rng, 8)

        @pl.when(wid < nw - 1)
        def _():
            pltpu.sync_copy(src_hbm.at[pl.ds(base, rng)],
                            out_hbm.at[pl.ds(base, rng)])

        @pl.when(wid == nw - 1)
        def _():
            pltpu.sync_copy(src_hbm.at[pl.ds(base, last_rng)],
                            out_hbm.at[pl.ds(base, last_rng)])

    @functools.partial(
        pl.kernel, mesh=mesh,
        out_type=(),
        scratch_types=[pltpu.VMEM((bpw,), jnp.int32),
                       pltpu.VMEM((bpw, MEM_DIM), jnp.float32),
                       pltpu.SemaphoreType.DMA],
    )
    def scatter_k(mem_hbm, idx_hbm, rows_hbm, idx_v, rows_v, sem):
        base = _wid() * bpw
        pltpu.sync_copy(idx_hbm.at[pl.ds(base, bpw)], idx_v)
        pltpu.sync_copy(rows_hbm.at[pl.ds(base, bpw)], rows_v)
        pltpu.async_copy(
            rows_v, mem_hbm.at[plsc.Indices(idx_v, ignored_value=-1)], sem
        ).wait()

    return gather_k, copy_k, scatter_k


def kernel(src_nodes, dst_nodes, edge_feat, timestamps, memory, last_update,
           W1, b1, W2, b2, W_ih, W_hh, b_ih, b_hh):
    gather_k, copy_k, scatter_k = _sc_kernels()

    nodes = jnp.concatenate([src_nodes, dst_nodes]).astype(jnp.int32)
    i = jnp.arange(B, dtype=jnp.int32)
    pos = jnp.concatenate([2 * i, 2 * i + 1])
    nodes_c = nodes.reshape(U, 1)
    nodes_r = nodes.reshape(1, U)
    pos_c = pos.reshape(U, 1)
    pos_r = pos.reshape(1, U)
    ts_col = jnp.tile(timestamps.astype(jnp.float32), 2).reshape(U, 1)

    edge_p = jnp.zeros((B, MEM_DIM), jnp.float32).at[:, :3].set(edge_feat)
    w1e = jnp.zeros((MEM_DIM, W1.shape[1]), jnp.float32).at[:3, :].set(W1[2 * MEM_DIM:])
    w1s = W1[:MEM_DIM]
    w1d = W1[MEM_DIM:2 * MEM_DIM]

    lu2d = jnp.zeros((LU_ROWS * MEM_DIM,), jnp.float32)
    lu2d = lu2d.at[:NUM_NODES].set(last_update.astype(jnp.float32))
    lu2d = lu2d.reshape(LU_ROWS, MEM_DIM)

    rows = gather_k(memory, nodes)

    out_rows, out_idx, out_lu = _tc_chain(
        rows, edge_p, ts_col, lu2d, nodes_c, nodes_r, pos_c, pos_r,
        w1s, w1d, w1e, b1.reshape(1, -1), W2, b2.reshape(1, -1),
        W_ih, b_ih.reshape(1, -1), W_hh, b_hh.reshape(1, -1))

    mem_ref = jax.new_ref(copy_k(memory))
    scatter_k(mem_ref, out_idx.reshape(U), out_rows)
    return mem_ref[...], out_lu.reshape(-1)[:NUM_NODES]


# speedup vs baseline: 22.2549x; 22.2549x over previous
import functools

import jax
import jax.numpy as jnp
from jax import lax
from jax.experimental import pallas as pl
from jax.experimental.pallas import tpu as pltpu
from jax.experimental.pallas import tpu_sc as plsc

NUM_NODES = 100000
MEM_DIM = 128
B = 1024
U = 2 * B
LU_ROWS = 784
POS_BITS = 11


def _dot(a, b, precision=jax.lax.Precision.DEFAULT):
    return jax.lax.dot_general(a, b, (((1,), (0,)), ((), ())),
                               precision=precision,
                               preferred_element_type=jnp.float32)


def _dot16(a, b):
    return _dot(a.astype(jnp.bfloat16), b.astype(jnp.bfloat16))


def _tc_chain_body(rows_ref, edge_ref, ts_ref, lu_ref, kc_ref, kr_ref,
                   w1s_ref, w1d_ref, w1e_ref, b1_ref,
                   w2_ref, b2_ref, wih_ref, bih_ref, whh_ref, bhh_ref,
                   out_rows_ref, out_idx_ref, out_lu_ref):
    rows = rows_ref[...]
    src_mem = rows[:B]
    dst_mem = rows[B:]

    pre = (_dot(src_mem, w1s_ref[...]) + _dot(dst_mem, w1d_ref[...])
           + _dot(edge_ref[...], w1e_ref[...]) + b1_ref[...])
    msg = _dot(jnp.maximum(pre, 0.0), w2_ref[...]) + b2_ref[...]

    gi = _dot(msg, wih_ref[...]) + bih_ref[...]
    gi2 = jnp.concatenate([gi, gi], axis=0)
    i_r = gi2[:, :MEM_DIM]
    i_z = gi2[:, MEM_DIM:2 * MEM_DIM]
    i_n = gi2[:, 2 * MEM_DIM:]

    k_c = kc_ref[...]
    k_r = kr_ref[...]
    nodes_c = jax.lax.shift_right_logical(k_c, POS_BITS)
    nodes_r = jax.lax.shift_right_logical(k_r, POS_BITS)

    big = jnp.int32(1 << 30)
    succ = jnp.where(k_c > k_r, k_c, big)
    next_k = jnp.min(succ, axis=0, keepdims=True)
    valid = jax.lax.shift_right_logical(next_k, POS_BITS) == nodes_r
    is_last_r = jnp.logical_not(valid)
    next_m = jnp.where(valid, next_k, -1)
    route = (k_c == next_m).astype(jnp.bfloat16)
    ones_u = jnp.ones((U, 1), jnp.bfloat16)
    incoming = _dot(route, ones_u)

    eq = nodes_c == nodes_r
    cnt = jnp.sum(eq.astype(jnp.int32), axis=0)
    rounds = jnp.max(cnt)

    whh = whh_ref[...].astype(jnp.bfloat16)
    bhh = bhh_ref[...]

    def gru(h):
        gh = _dot(h.astype(jnp.bfloat16), whh) + bhh
        h_r = gh[:, :MEM_DIM]
        h_z = gh[:, MEM_DIM:2 * MEM_DIM]
        h_n = gh[:, 2 * MEM_DIM:]
        r = jax.nn.sigmoid(i_r + h_r)
        z = jax.nn.sigmoid(i_z + h_z)
        n = jnp.tanh(i_n + r * h_n)
        return (1.0 - z) * n + z * h

    def body(_, h):
        g = gru(h)
        return _dot16(route, g) + (1.0 - incoming) * h

    h = lax.fori_loop(0, rounds - 1, body, rows)
    g = gru(h)
    out_rows_ref[...] = g
    out_idx_ref[...] = jnp.where(is_last_r, nodes_r, -1)

    off = jnp.float32(2048.0)
    lane = lax.broadcasted_iota(jnp.int32, (1, MEM_DIM), 1)
    lane_oh = (lane == (nodes_c % MEM_DIM)).astype(jnp.float32)
    rowid = lax.broadcasted_iota(jnp.int32, (LU_ROWS, 1), 0)
    q_row = nodes_r // MEM_DIM
    rowpick = jnp.where((rowid == q_row) & is_last_r, 1.0, 0.0)
    marked = _dot(rowpick, lane_oh * (ts_ref[...] + off),
                  precision=jax.lax.Precision.HIGHEST)
    out_lu_ref[...] = jnp.where(marked > off - 1.0, marked - off, lu_ref[...])


def _tc_chain(rows, edge_p, ts_col, lu2d, k_c, k_r,
              w1s, w1d, w1e, b1, w2, b2, wih, bih, whh, bhh):
    return pl.pallas_call(
        _tc_chain_body,
        out_shape=(jax.ShapeDtypeStruct((U, MEM_DIM), jnp.float32),
                   jax.ShapeDtypeStruct((1, U), jnp.int32),
                   jax.ShapeDtypeStruct((LU_ROWS, MEM_DIM), jnp.float32)),
    )(rows, edge_p, ts_col, lu2d, k_c, k_r,
      w1s, w1d, w1e, b1, w2, b2, wih, bih, whh, bhh)


@functools.lru_cache(maxsize=None)
def _sc_kernels():
    mesh = plsc.VectorSubcoreMesh(core_axis_name="c", subcore_axis_name="s")
    nc = mesh.num_cores
    nw = nc * mesh.num_subcores
    bpw = U // nw

    def _wid():
        return lax.axis_index("s") * nc + lax.axis_index("c")

    @functools.partial(
        pl.kernel, mesh=mesh,
        out_type=jax.ShapeDtypeStruct((U, MEM_DIM), jnp.float32),
        scratch_types=[pltpu.VMEM((bpw,), jnp.int32),
                       pltpu.VMEM((bpw, MEM_DIM), jnp.float32),
                       pltpu.SemaphoreType.DMA],
    )
    def gather_k(table_hbm, idx_hbm, out_hbm, idx_v, rows_v, sem):
        base = _wid() * bpw
        pltpu.sync_copy(idx_hbm.at[pl.ds(base, bpw)], idx_v)
        pltpu.async_copy(table_hbm.at[idx_v], rows_v, sem).wait()
        pltpu.sync_copy(rows_v, out_hbm.at[pl.ds(base, bpw)])

    rng = 3128
    last_rng = NUM_NODES - (nw - 1) * rng
    ch = 392

    def _copy_range(src_hbm, out_hbm, base, size, bufs, sem_i, sem_o):
        sizes = []
        off = 0
        while off < size:
            c = min(ch, size - off)
            sizes.append((off, c))
            off += c
        n = len(sizes)
        ins = [None] * n
        outs = [None] * n

        def start_in(k):
            off, c = sizes[k]
            return pltpu.async_copy(src_hbm.at[pl.ds(base + off, c)],
                                    bufs[k % 2].at[pl.ds(0, c)], sem_i[k % 2])

        ins[0] = start_in(0)
        for k in range(n):
            off, c = sizes[k]
            ins[k].wait()
            outs[k] = pltpu.async_copy(bufs[k % 2].at[pl.ds(0, c)],
                                       out_hbm.at[pl.ds(base + off, c)],
                                       sem_o[k % 2])
            if k + 1 < n:
                if k - 1 >= 0:
                    outs[k - 1].wait()
                ins[k + 1] = start_in(k + 1)
        if n >= 2:
            outs[n - 2].wait()
        outs[n - 1].wait()

    @functools.partial(
        pl.kernel, mesh=mesh,
        out_type=jax.ShapeDtypeStruct((NUM_NODES, MEM_DIM), jnp.float32),
        scratch_types=[pltpu.VMEM((ch, MEM_DIM), jnp.float32),
                       pltpu.VMEM((ch, MEM_DIM), jnp.float32),
                       pltpu.SemaphoreType.DMA, pltpu.SemaphoreType.DMA,
                       pltpu.SemaphoreType.DMA, pltpu.SemaphoreType.DMA],
    )
    def copy_k(src_hbm, out_hbm, buf_a, buf_b, si0, si1, so0, so1):
        wid = _wid()
        base = pl.multiple_of(wid * rng, 8)

        @pl.when(wid < nw - 1)
        def _():
            _copy_range(src_hbm, out_hbm, base, rng,
                        (buf_a, buf_b), (si0, si1), (so0, so1))

        @pl.when(wid == nw - 1)
        def _():
            _copy_range(src_hbm, out_hbm, base, last_rng,
                        (buf_a, buf_b), (si0, si1), (so0, so1))

    @functools.partial(
        pl.kernel, mesh=mesh,
        out_type=(),
        scratch_types=[pltpu.VMEM((bpw,), jnp.int32),
                       pltpu.VMEM((bpw, MEM_DIM), jnp.float32),
                       pltpu.SemaphoreType.DMA],
    )
    def scatter_k(mem_hbm, idx_hbm, rows_hbm, idx_v, rows_v, sem):
        base = _wid() * bpw
        pltpu.sync_copy(idx_hbm.at[pl.ds(base, bpw)], idx_v)
        pltpu.sync_copy(rows_hbm.at[pl.ds(base, bpw)], rows_v)
        pltpu.async_copy(
            rows_v, mem_hbm.at[plsc.Indices(idx_v, ignored_value=-1)], sem
        ).wait()

    return gather_k, copy_k, scatter_k


def kernel(src_nodes, dst_nodes, edge_feat, timestamps, memory, last_update,
           W1, b1, W2, b2, W_ih, W_hh, b_ih, b_hh):
    gather_k, copy_k, scatter_k = _sc_kernels()

    nodes = jnp.concatenate([src_nodes, dst_nodes]).astype(jnp.int32)
    i = jnp.arange(B, dtype=jnp.int32)
    pos = jnp.concatenate([2 * i, 2 * i + 1])
    keys = nodes * (1 << POS_BITS) + pos
    ts_col = jnp.tile(timestamps.astype(jnp.float32), 2).reshape(U, 1)

    edge_p = jnp.zeros((B, MEM_DIM), jnp.float32).at[:, :3].set(edge_feat)
    w1e = jnp.zeros((MEM_DIM, W1.shape[1]), jnp.float32).at[:3, :].set(W1[2 * MEM_DIM:])
    w1s = W1[:MEM_DIM]
    w1d = W1[MEM_DIM:2 * MEM_DIM]

    lu2d = jnp.zeros((LU_ROWS * MEM_DIM,), jnp.float32)
    lu2d = lu2d.at[:NUM_NODES].set(last_update.astype(jnp.float32))
    lu2d = lu2d.reshape(LU_ROWS, MEM_DIM)

    mem_copy = copy_k(memory)
    rows = gather_k(memory, nodes)

    out_rows, out_idx, out_lu = _tc_chain(
        rows, edge_p, ts_col, lu2d, keys.reshape(U, 1), keys.reshape(1, U),
        w1s, w1d, w1e, b1.reshape(1, -1), W2, b2.reshape(1, -1),
        W_ih, b_ih.reshape(1, -1), W_hh, b_hh.reshape(1, -1))

    mem_ref = jax.new_ref(mem_copy)
    scatter_k(mem_ref, out_idx.reshape(U), out_rows)
    return mem_ref[...], out_lu.reshape(-1)[:NUM_NODES]
